# baseline (speedup 1.0000x reference)
import functools
import math

import jax
import jax.numpy as jnp
from jax import lax
from jax.experimental import pallas as pl
from jax.experimental.pallas import tpu as pltpu
from jax.experimental.pallas import tpu_sc as plsc

_N = 2048
_EMBED = 768
_HEADS = 12
_HEAD_DIM = _EMBED // _HEADS
_NUM_SAMPLES = 256
_SCALE = 1.0 / math.sqrt(float(_N))
_NEG = -1e30

_NW = 32
_ROWS_PER_W = _N // _NW
_CHUNK = 16
_G = _NUM_SAMPLES // 16


def _mask_body(samples_ref, mask_ref, idx_v, buf_v):
    wid = lax.axis_index("s") * 2 + lax.axis_index("c")
    base = wid * _ROWS_PER_W
    pltpu.sync_copy(
        samples_ref.at[pl.ds(base * _NUM_SAMPLES, _ROWS_PER_W * _NUM_SAMPLES)],
        idx_v,
    )
    neg = jnp.full((16,), _NEG, jnp.float32)
    zero = jnp.zeros((16,), jnp.float32)

    def fill(i, carry):
        for g in range(_N // 16):
            buf_v[i, pl.ds(g * 16, 16)] = neg
        return carry

    lax.fori_loop(0, _CHUNK, fill, 0)

    n_chunks = _ROWS_PER_W // _CHUNK
    for c in range(n_chunks):
        def scatter_row(r, carry, _c=c, _val=zero):
            row = jnp.full((16,), 0, jnp.int32) + r
            samp_off = (_c * _CHUNK + r) * _NUM_SAMPLES
            for g in range(_G):
                idx = idx_v[pl.ds(samp_off + g * 16, 16)]
                plsc.store_scatter(buf_v, [row, idx], _val)
            return carry

        lax.fori_loop(0, _CHUNK, scatter_row, 0)
        pltpu.sync_copy(
            buf_v, mask_ref.at[pl.ds(base + c * _CHUNK, _CHUNK)]
        )

        if c + 1 < n_chunks:
            lax.fori_loop(
                0, _CHUNK, functools.partial(scatter_row, _val=neg), 0
            )


@functools.cache
def _get_mask_builder():
    return pl.kernel(
        _mask_body,
        out_type=jax.ShapeDtypeStruct((_N, _N), jnp.float32),
        mesh=plsc.VectorSubcoreMesh(core_axis_name="c", subcore_axis_name="s"),
        scratch_types=[
            pltpu.VMEM((_ROWS_PER_W * _NUM_SAMPLES,), jnp.int32),
            pltpu.VMEM((_CHUNK, _N), jnp.float32),
        ],
        compiler_params=pltpu.CompilerParams(
            needs_layout_passes=False, use_tc_tiling_on_sc=False
        ),
    )


def _build_mask(samples):
    return _get_mask_builder()(samples.reshape(-1))


_BR = 512
_NSPLIT = 2
_DN_T = (((1,), (1,)), ((), ()))
_DN_N = (((1,), (0,)), ((), ()))


def _attn_body(q_in_ref, wq_ref, bq_ref, key_ref, wk_ref, bk_ref,
               value_ref, wv_ref, bv_ref, mask_ref, wo_ref, bo_ref,
               out_ref, k_s, v_s, kn_s):
    r = pl.program_id(0)
    h = pl.program_id(1)

    @pl.when(r == 0)
    def _project_kv():
        k = lax.dot_general(
            key_ref[...], wk_ref[0], _DN_T, preferred_element_type=jnp.float32
        ) + bk_ref[0]
        kn_s[h, 0] = jnp.sqrt(jnp.max(jnp.sum(k * k, axis=1)))
        k_s[h] = k.astype(jnp.bfloat16)
        v = (
            lax.dot_general(
                value_ref[...], wv_ref[0], _DN_T,
                preferred_element_type=jnp.float32,
            ) + bv_ref[0]
        ).astype(jnp.bfloat16)
        col = lax.broadcasted_iota(jnp.int32, (_N, _HEAD_DIM), 1)
        pad = jnp.where(col == 0, 1.0, 0.0).astype(jnp.bfloat16)
        v_s[h] = jnp.concatenate([v, pad], axis=1)

    q = lax.dot_general(
        q_in_ref[...], wq_ref[0], _DN_T, preferred_element_type=jnp.float32
    ) + bq_ref[0]
    qn = jnp.sqrt(jnp.sum(q * q, axis=1, keepdims=True))
    m = qn * kn_s[h, 0]
    qb = q.astype(jnp.bfloat16)
    nc = _N // _NSPLIT
    av = jnp.zeros((_BR, 2 * _HEAD_DIM), jnp.float32)
    for t in range(_NSPLIT):
        sl = pl.ds(t * nc, nc)
        logits = lax.dot_general(
            qb, k_s[h, sl], _DN_T, preferred_element_type=jnp.float32
        )
        e = jnp.exp(logits - m + mask_ref[:, sl]).astype(jnp.bfloat16)
        av = av + lax.dot_general(
            e, v_s[h, sl], _DN_N, preferred_element_type=jnp.float32
        )
    attn = av[:, :_HEAD_DIM] / av[:, _HEAD_DIM:_HEAD_DIM + 1]
    o = lax.dot_general(
        attn.astype(jnp.bfloat16), wo_ref[0], _DN_N,
        preferred_element_type=jnp.float32,
    )

    @pl.when(h == 0)
    def _():
        out_ref[...] = o + bo_ref[...]

    @pl.when(h != 0)
    def _():
        out_ref[...] += o


def _head_spec():
    return pl.BlockSpec((1, _HEAD_DIM, _EMBED), lambda r, h: (h, 0, 0))


def _bias_spec():
    return pl.BlockSpec((1, 1, _HEAD_DIM), lambda r, h: (h, 0, 0))


_attn = pl.pallas_call(
    _attn_body,
    grid=(_N // _BR, _HEADS),
    in_specs=[
        pl.BlockSpec((_BR, _EMBED), lambda r, h: (r, 0)),
        _head_spec(),
        _bias_spec(),
        pl.BlockSpec((_N, _EMBED), lambda r, h: (0, 0)),
        _head_spec(),
        _bias_spec(),
        pl.BlockSpec((_N, _EMBED), lambda r, h: (0, 0)),
        _head_spec(),
        _bias_spec(),
        pl.BlockSpec((_BR, _N), lambda r, h: (r, 0)),
        _head_spec(),
        pl.BlockSpec((1, _EMBED), lambda r, h: (0, 0)),
    ],
    out_specs=pl.BlockSpec((_BR, _EMBED), lambda r, h: (r, 0)),
    out_shape=jax.ShapeDtypeStruct((_N, _EMBED), jnp.float32),
    scratch_shapes=[
        pltpu.VMEM((_HEADS, _N, _HEAD_DIM), jnp.bfloat16),
        pltpu.VMEM((_HEADS, _N, 2 * _HEAD_DIM), jnp.bfloat16),
        pltpu.SMEM((_HEADS, 1), jnp.float32),
    ],
    compiler_params=pltpu.CompilerParams(
        vmem_limit_bytes=100 * 1024 * 1024,
    ),
)


def kernel(query, key, value, Wq, bq, Wk, bk, Wv, bv, Wo, bo, samples):
    bf = jnp.bfloat16
    mask = _build_mask(samples)
    wk3 = Wk.reshape(_HEADS, _HEAD_DIM, _EMBED).astype(bf)
    wv3 = Wv.reshape(_HEADS, _HEAD_DIM, _EMBED).astype(bf)
    wq3 = (Wq * _SCALE).reshape(_HEADS, _HEAD_DIM, _EMBED).astype(bf)
    wo3 = Wo.T.reshape(_HEADS, _HEAD_DIM, _EMBED).astype(bf)
    bk3 = bk.reshape(_HEADS, 1, _HEAD_DIM)
    bv3 = bv.reshape(_HEADS, 1, _HEAD_DIM)
    bq3 = (bq * _SCALE).reshape(_HEADS, 1, _HEAD_DIM)
    out = _attn(
        query.astype(bf), wq3, bq3,
        key.astype(bf), wk3, bk3,
        value.astype(bf), wv3, bv3,
        mask, wo3, bo.reshape(1, _EMBED),
    )
    return out.reshape(_N, 1, _EMBED)


# speedup vs baseline: 1.2880x; 1.2880x over previous
import functools
import math

import jax
import jax.numpy as jnp
from jax import lax
from jax.experimental import pallas as pl
from jax.experimental.pallas import tpu as pltpu
from jax.experimental.pallas import tpu_sc as plsc

_N = 2048
_EMBED = 768
_HEADS = 12
_HEAD_DIM = _EMBED // _HEADS
_NUM_SAMPLES = 256
_SCALE = 1.0 / math.sqrt(float(_N))
_NEG = -1e30

_NW = 32
_ROWS_PER_W = _N // _NW
_CHUNK = 16
_G = _NUM_SAMPLES // 16


def _mask_body(samples_ref, mask_ref, idx_v, buf_v):
    wid = lax.axis_index("s") * 2 + lax.axis_index("c")
    base = wid * _ROWS_PER_W
    pltpu.sync_copy(
        samples_ref.at[pl.ds(base * _NUM_SAMPLES, _ROWS_PER_W * _NUM_SAMPLES)],
        idx_v,
    )
    neg = jnp.full((16,), _NEG, jnp.float32)
    zero = jnp.zeros((16,), jnp.float32)

    def fill(i, carry):
        for g in range(_N // 16):
            buf_v[i, pl.ds(g * 16, 16)] = neg
        return carry

    lax.fori_loop(0, _CHUNK, fill, 0)

    n_chunks = _ROWS_PER_W // _CHUNK
    for c in range(n_chunks):
        def scatter_row(r, carry, _c=c, _val=zero):
            row = jnp.full((16,), 0, jnp.int32) + r
            samp_off = (_c * _CHUNK + r) * _NUM_SAMPLES
            for g in range(_G):
                idx = idx_v[pl.ds(samp_off + g * 16, 16)]
                plsc.store_scatter(buf_v, [row, idx], _val)
            return carry

        lax.fori_loop(0, _CHUNK, scatter_row, 0)
        pltpu.sync_copy(
            buf_v, mask_ref.at[pl.ds(base + c * _CHUNK, _CHUNK)]
        )

        if c + 1 < n_chunks:
            lax.fori_loop(
                0, _CHUNK, functools.partial(scatter_row, _val=neg), 0
            )


@functools.cache
def _get_mask_builder():
    return pl.kernel(
        _mask_body,
        out_type=jax.ShapeDtypeStruct((_N, _N), jnp.float32),
        mesh=plsc.VectorSubcoreMesh(core_axis_name="c", subcore_axis_name="s"),
        scratch_types=[
            pltpu.VMEM((_ROWS_PER_W * _NUM_SAMPLES,), jnp.int32),
            pltpu.VMEM((_CHUNK, _N), jnp.float32),
        ],
        compiler_params=pltpu.CompilerParams(
            needs_layout_passes=False, use_tc_tiling_on_sc=False
        ),
    )


def _build_mask(samples):
    return _get_mask_builder()(samples.reshape(-1))


_BR = 512
_NSPLIT = 2
_DN_T = (((1,), (1,)), ((), ()))
_DN_N = (((1,), (0,)), ((), ()))


def _attn_body(q_in_ref, wq_ref, bq_ref, key_ref, wk_ref, bk_ref,
               value_ref, wv_ref, bv_ref, mask_ref, wo_ref, bo_ref,
               out_ref, k_s, v_s, kn_s, q_s, attn_s):
    r = pl.program_id(0)
    h = pl.program_id(1)

    @pl.when(r == 0)
    def _project_kv():
        k = lax.dot_general(
            key_ref[...], wk_ref[0], _DN_T, preferred_element_type=jnp.float32
        ) + bk_ref[0]
        kn_s[h, 0] = jnp.sqrt(jnp.max(jnp.sum(k * k, axis=1)))
        k_s[h] = k.astype(jnp.bfloat16)
        v = (
            lax.dot_general(
                value_ref[...], wv_ref[0], _DN_T,
                preferred_element_type=jnp.float32,
            ) + bv_ref[0]
        ).astype(jnp.bfloat16)
        col = lax.broadcasted_iota(jnp.int32, (_N, _HEAD_DIM), 1)
        pad = jnp.where(col == 0, 1.0, 0.0).astype(jnp.bfloat16)
        v_s[h] = jnp.concatenate([v, pad], axis=1)

    @pl.when(h == 0)
    def _project_q():
        q_all = lax.dot_general(
            q_in_ref[...], wq_ref[...], _DN_T,
            preferred_element_type=jnp.float32,
        ) + bq_ref[...]
        for h2 in range(_HEADS):
            q_s[h2] = q_all[:, h2 * _HEAD_DIM:(h2 + 1) * _HEAD_DIM].astype(
                jnp.bfloat16
            )

    qb = q_s[h]
    qf = qb.astype(jnp.float32)
    qn = jnp.sqrt(jnp.sum(qf * qf, axis=1, keepdims=True))
    m = qn * kn_s[h, 0]
    logits = lax.dot_general(
        qb, k_s[h], _DN_T, preferred_element_type=jnp.float32
    )
    e = jnp.exp(logits - m + mask_ref[...]).astype(jnp.bfloat16)
    av = lax.dot_general(
        e, v_s[h], _DN_N, preferred_element_type=jnp.float32
    )
    attn_s[h] = (
        av[:, :_HEAD_DIM] / av[:, _HEAD_DIM:_HEAD_DIM + 1]
    ).astype(jnp.bfloat16)

    @pl.when(h == _HEADS - 1)
    def _project_out():
        attn_all = jnp.concatenate(
            [attn_s[h2] for h2 in range(_HEADS)], axis=1
        )
        out_ref[...] = lax.dot_general(
            attn_all, wo_ref[...], _DN_T, preferred_element_type=jnp.float32
        ) + bo_ref[...]


def _head_spec():
    return pl.BlockSpec((1, _HEAD_DIM, _EMBED), lambda r, h: (h, 0, 0))


def _bias_spec():
    return pl.BlockSpec((1, 1, _HEAD_DIM), lambda r, h: (h, 0, 0))


_attn = pl.pallas_call(
    _attn_body,
    grid=(_N // _BR, _HEADS),
    in_specs=[
        pl.BlockSpec((_BR, _EMBED), lambda r, h: (r, 0)),
        pl.BlockSpec((_EMBED, _EMBED), lambda r, h: (0, 0)),
        pl.BlockSpec((1, _EMBED), lambda r, h: (0, 0)),
        pl.BlockSpec((_N, _EMBED), lambda r, h: (0, 0)),
        _head_spec(),
        _bias_spec(),
        pl.BlockSpec((_N, _EMBED), lambda r, h: (0, 0)),
        _head_spec(),
        _bias_spec(),
        pl.BlockSpec((_BR, _N), lambda r, h: (r, 0)),
        pl.BlockSpec((_EMBED, _EMBED), lambda r, h: (0, 0)),
        pl.BlockSpec((1, _EMBED), lambda r, h: (0, 0)),
    ],
    out_specs=pl.BlockSpec((_BR, _EMBED), lambda r, h: (r, 0)),
    out_shape=jax.ShapeDtypeStruct((_N, _EMBED), jnp.float32),
    scratch_shapes=[
        pltpu.VMEM((_HEADS, _N, _HEAD_DIM), jnp.bfloat16),
        pltpu.VMEM((_HEADS, _N, 2 * _HEAD_DIM), jnp.bfloat16),
        pltpu.SMEM((_HEADS, 1), jnp.float32),
        pltpu.VMEM((_HEADS, _BR, _HEAD_DIM), jnp.bfloat16),
        pltpu.VMEM((_HEADS, _BR, _HEAD_DIM), jnp.bfloat16),
    ],
    compiler_params=pltpu.CompilerParams(
        vmem_limit_bytes=100 * 1024 * 1024,
    ),
)


def kernel(query, key, value, Wq, bq, Wk, bk, Wv, bv, Wo, bo, samples):
    bf = jnp.bfloat16
    mask = _build_mask(samples)
    wk3 = Wk.reshape(_HEADS, _HEAD_DIM, _EMBED).astype(bf)
    wv3 = Wv.reshape(_HEADS, _HEAD_DIM, _EMBED).astype(bf)
    bk3 = bk.reshape(_HEADS, 1, _HEAD_DIM)
    bv3 = bv.reshape(_HEADS, 1, _HEAD_DIM)
    out = _attn(
        query.astype(bf), (Wq * _SCALE).astype(bf),
        (bq * _SCALE).reshape(1, _EMBED),
        key.astype(bf), wk3, bk3,
        value.astype(bf), wv3, bv3,
        mask, Wo.astype(bf), bo.reshape(1, _EMBED),
    )
    return out.reshape(_N, 1, _EMBED)
